# baseline (device time: 34441 ns/iter reference)
import jax
import jax.numpy as jnp
from jax import lax
from jax.experimental import pallas as pl
from jax.experimental.pallas import tpu as pltpu

M = 1024
N = 1024
D = 4096
H = M // 2
K_CMP = 2
K_COM = 8
CC = H // K_CMP
R = H // K_COM


def kernel(dy, W):
    def body(dy_ref, w_ref, out_ref, wvmem, dybuf, pbuf, ybuf,
             w_sem, dy_sems, out_sems,
             ysend_sems, yrecv_sems, xsend_sems, xrecv_sems):
        my_x = lax.axis_index("x")
        my_y = lax.axis_index("y")

        barrier_sem = pltpu.get_barrier_semaphore()
        pl.semaphore_signal(
            barrier_sem, inc=1,
            device_id=(my_x, 1 - my_y), device_id_type=pl.DeviceIdType.MESH)
        pl.semaphore_signal(
            barrier_sem, inc=1,
            device_id=(1 - my_x, my_y), device_id_type=pl.DeviceIdType.MESH)

        row0 = my_x * H

        w_load = pltpu.make_async_copy(w_ref, wvmem, w_sem)
        w_load.start()

        def dy_load(c):
            return pltpu.make_async_copy(
                dy_ref.at[pl.ds(row0 + c * CC, CC)],
                dybuf.at[c],
                dy_sems.at[c],
            )

        for c in range(K_CMP):
            dy_load(c).start()

        def y_copy(k):
            return pltpu.make_async_remote_copy(
                src_ref=pbuf.at[pl.ds(k * R, R)],
                dst_ref=ybuf.at[pl.ds(k * R, R)],
                send_sem=ysend_sems.at[k],
                recv_sem=yrecv_sems.at[k],
                device_id=(my_x, 1 - my_y),
                device_id_type=pl.DeviceIdType.MESH,
            )

        def x_copy(k):
            return pltpu.make_async_remote_copy(
                src_ref=pbuf.at[pl.ds(k * R, R)],
                dst_ref=out_ref.at[pl.ds(row0 + k * R, R)],
                send_sem=xsend_sems.at[k],
                recv_sem=xrecv_sems.at[k],
                device_id=(1 - my_x, my_y),
                device_id_type=pl.DeviceIdType.MESH,
            )

        def out_copy(k):
            return pltpu.make_async_copy(
                pbuf.at[pl.ds(k * R, R)],
                out_ref.at[pl.ds(row0 + k * R, R)],
                out_sems.at[k],
            )

        w_load.wait()
        for c in range(K_CMP):
            dy_load(c).wait()
            p = lax.dot_general(
                dybuf[c], wvmem[...],
                dimension_numbers=(((1,), (1,)), ((), ())),
                preferred_element_type=jnp.float32,
            )
            pbuf[pl.ds(c * CC, CC), :] = p.astype(jnp.bfloat16)
            if c == 0:
                pl.semaphore_wait(barrier_sem, 2)
            for s in range(K_COM // K_CMP):
                y_copy(c * (K_COM // K_CMP) + s).start()

        for k in range(K_COM):
            yc = y_copy(k)
            yc.wait_send()
            yc.wait_recv()
            pbuf[pl.ds(k * R, R), :] = (
                pbuf[pl.ds(k * R, R), :] + ybuf[pl.ds(k * R, R), :]
            )
            out_copy(k).start()
            x_copy(k).start()

        for k in range(K_COM):
            x_copy(k).wait()
            out_copy(k).wait()

    return pl.pallas_call(
        body,
        out_shape=jax.ShapeDtypeStruct((M, N), jnp.bfloat16),
        in_specs=[
            pl.BlockSpec(memory_space=pl.ANY),
            pl.BlockSpec(memory_space=pl.ANY),
        ],
        out_specs=pl.BlockSpec(memory_space=pl.ANY),
        scratch_shapes=[
            pltpu.VMEM((M, D), jnp.float32),
            pltpu.VMEM((K_CMP, CC, D), jnp.float32),
            pltpu.VMEM((H, N), jnp.bfloat16),
            pltpu.VMEM((H, N), jnp.bfloat16),
            pltpu.SemaphoreType.DMA,
            pltpu.SemaphoreType.DMA((K_CMP,)),
            pltpu.SemaphoreType.DMA((K_COM,)),
            pltpu.SemaphoreType.DMA((K_COM,)),
            pltpu.SemaphoreType.DMA((K_COM,)),
            pltpu.SemaphoreType.DMA((K_COM,)),
            pltpu.SemaphoreType.DMA((K_COM,)),
        ],
        compiler_params=pltpu.CompilerParams(collective_id=0),
    )(dy, W)
